# Initial kernel scaffold; baseline (speedup 1.0000x reference)
import jax
import jax.numpy as jnp
from jax.experimental import pallas as pl


def kernel(X_v, target_X_v, weights):
    raise NotImplementedError("write your pallas kernel here")


# speedup vs baseline: 1.6814x; 1.6814x over previous
import jax
import jax.numpy as jnp
from jax.experimental import pallas as pl
from jax.experimental.pallas import tpu as pltpu


def _sl1(a, b):
    d = a - b
    ad = jnp.abs(d)
    return jnp.where(ad < 1.0, 0.5 * d * d, ad - 0.5)


def _body(xT_ref, tT_ref, w_ref, loss_ref, lossc_ref):
    i = pl.program_id(0)
    xT = xT_ref[0]
    tT = tT_ref[0]
    N = xT.shape[1]
    M = tT.shape[1]
    G = jax.lax.dot_general(xT, tT, (((0,), (0,)), ((), ())),
                            preferred_element_type=jnp.float32)
    c = jnp.sum(tT * tT, axis=0)
    D = c[None, :] - (G + G)
    minD = jnp.min(D, axis=1)
    iota_m = jax.lax.broadcasted_iota(jnp.int32, (N, M), 1)
    matches = D <= minD[:, None]
    idx = jnp.min(jnp.where(matches, iota_m, M), axis=1)
    ohT = (jax.lax.broadcasted_iota(jnp.int32, (M, N), 0) == idx[None, :]
           ).astype(jnp.float32)
    tagpT = jax.lax.dot_general(tT, ohT, (((1,), (0,)), ((), ())),
                                preferred_element_type=jnp.float32)
    w = w_ref[0, 0, 0]
    part_loss = jnp.sum(_sl1(xT, tagpT)) / (N * 3.0) * w / 4.0
    sx = jnp.sum(xT, axis=1) / N
    st = jnp.sum(tT, axis=1) / M
    part_lossc = jnp.sum(_sl1(sx, st)) / 12.0

    @pl.when(i == 0)
    def _():
        loss_ref[...] = jnp.zeros((1, 1), jnp.float32)
        lossc_ref[...] = jnp.zeros((1, 1), jnp.float32)

    loss_ref[...] = loss_ref[...] + part_loss
    lossc_ref[...] = lossc_ref[...] + part_lossc


def kernel(X_v, target_X_v, weights):
    B, K, N, D = X_v.shape
    P = B * K
    xT = jnp.transpose(X_v, (0, 1, 3, 2)).reshape(P, D, N)
    tT = jnp.transpose(target_X_v, (0, 1, 3, 2)).reshape(P, D, N)
    w = weights.reshape(P, 1, 1)
    loss, lossc = pl.pallas_call(
        _body,
        grid=(P,),
        in_specs=[
            pl.BlockSpec((1, D, N), lambda i: (i, 0, 0)),
            pl.BlockSpec((1, D, N), lambda i: (i, 0, 0)),
            pl.BlockSpec((1, 1, 1), lambda i: (i, 0, 0)),
        ],
        out_specs=[
            pl.BlockSpec((1, 1), lambda i: (0, 0)),
            pl.BlockSpec((1, 1), lambda i: (0, 0)),
        ],
        out_shape=[
            jax.ShapeDtypeStruct((1, 1), jnp.float32),
            jax.ShapeDtypeStruct((1, 1), jnp.float32),
        ],
        compiler_params=pltpu.CompilerParams(
            dimension_semantics=("arbitrary",),
        ),
    )(xT, tT, w)
    return loss[0, 0], lossc[0, 0]
